# baseline (speedup 1.0000x reference)
import jax
import jax.numpy as jnp
from jax import lax
from jax.experimental import pallas as pl
from jax.experimental.pallas import tpu as pltpu
from jax.experimental.pallas import tpu_sc as plsc

NUM_INPUTS = 100000
D = 128
B = 16384
K = 50

NC = 2
NS = 16
NW = NC * NS
RPW = B // NW
BLK = 128
NBLK = RPW // BLK
LANES = 16
DCH = D // LANES

_BCAST_DNUMS = lax.GatherDimensionNumbers(
    offset_dims=(), collapsed_slice_dims=(0,), start_index_map=(0,))


def _lane_broadcast(vec, lane):
    idx = jnp.full((LANES, 1), lane, dtype=jnp.int32)
    return lax.gather(vec, idx, _BCAST_DNUMS, (1,),
                      mode=lax.GatherScatterMode.PROMISE_IN_BOUNDS)


def _sc_body(idx_hbm, vals_hbm, weight_hbm, bias_hbm, out_hbm,
             idx_v, vals_v, rows_v, acc_v, bias_v, gsem, ssem, osem):
    wid = lax.axis_index("s") * NC + lax.axis_index("c")
    col0 = wid * RPW

    pltpu.sync_copy(bias_hbm, bias_v)
    pltpu.sync_copy(idx_hbm.at[:, pl.ds(col0, BLK)], idx_v.at[0])
    pltpu.sync_copy(vals_hbm.at[:, pl.ds(col0, BLK)], vals_v.at[0])

    for blk in range(NBLK):
        sb = blk % 2
        base = col0 + blk * BLK

        def fire_gather(kk, buf, sb=sb):
            pltpu.async_copy(weight_hbm.at[idx_v.at[sb, kk]],
                             rows_v.at[buf], gsem.at[buf])

        def wait_gather(kk, buf, sb=sb):
            pltpu.make_async_copy(weight_hbm.at[idx_v.at[sb, kk]],
                                  rows_v.at[buf], gsem.at[buf]).wait()

        if blk >= 1:
            pltpu.make_async_copy(idx_hbm.at[:, pl.ds(base, BLK)],
                                  idx_v.at[sb], ssem.at[0]).wait()
            pltpu.make_async_copy(vals_hbm.at[:, pl.ds(base, BLK)],
                                  vals_v.at[sb], ssem.at[1]).wait()

        fire_gather(0, 0)

        if blk + 1 < NBLK:
            pltpu.async_copy(idx_hbm.at[:, pl.ds(base + BLK, BLK)],
                             idx_v.at[1 - sb], ssem.at[0])
            pltpu.async_copy(vals_hbm.at[:, pl.ds(base + BLK, BLK)],
                             vals_v.at[1 - sb], ssem.at[1])

        if blk >= 2:
            pltpu.make_async_copy(acc_v.at[sb],
                                  out_hbm.at[pl.ds(base - 2 * BLK, BLK)],
                                  osem.at[sb]).wait()

        bias_c = tuple(bias_v[pl.ds(j * LANES, LANES)] for j in range(DCH))

        @pl.loop(0, BLK)
        def _init(rr, sb=sb, bias_c=bias_c):
            for j in range(DCH):
                acc_v[sb, rr, pl.ds(j * LANES, LANES)] = bias_c[j]

        @pl.loop(0, K, step=2)
        def _k_loop(k, sb=sb, fire_gather=fire_gather,
                    wait_gather=wait_gather):
            for p in range(2):
                kk = k + p

                @pl.when(kk + 1 < K)
                def _():
                    fire_gather(kk + 1, 1 - p)

                wait_gather(kk, p)

                @pl.loop(0, BLK // LANES)
                def _bc(bc, kk=kk, p=p, sb=sb):
                    off = pl.multiple_of(bc * LANES, LANES)
                    vb16 = vals_v[sb, kk, pl.ds(off, LANES)]
                    for i in range(LANES):
                        vb = _lane_broadcast(vb16, i)
                        row = off + i
                        for j in range(DCH):
                            plsc.addupdate(
                                acc_v.at[sb, row, pl.ds(j * LANES, LANES)],
                                rows_v[p, row, pl.ds(j * LANES, LANES)] * vb)

        pltpu.async_copy(acc_v.at[sb], out_hbm.at[pl.ds(base, BLK)],
                         osem.at[sb])

    for sb in range(2):
        pltpu.make_async_copy(acc_v.at[sb], out_hbm.at[pl.ds(col0, BLK)],
                              osem.at[sb]).wait()


@jax.jit
def kernel(feature_indices, feature_values, weight, bias):
    idx_t = feature_indices.T
    vals_t = feature_values.T

    mesh = plsc.VectorSubcoreMesh(core_axis_name="c", subcore_axis_name="s")
    run = pl.kernel(
        _sc_body,
        out_type=jax.ShapeDtypeStruct((B, D), jnp.float32),
        mesh=mesh,
        scratch_types=[
            pltpu.VMEM((2, K, BLK), jnp.int32),
            pltpu.VMEM((2, K, BLK), jnp.float32),
            pltpu.VMEM((2, BLK, D), jnp.float32),
            pltpu.VMEM((2, BLK, D), jnp.float32),
            pltpu.VMEM((D,), jnp.float32),
            pltpu.SemaphoreType.DMA((2,)),
            pltpu.SemaphoreType.DMA((2,)),
            pltpu.SemaphoreType.DMA((2,)),
        ],
    )
    return run(idx_t, vals_t, weight, bias)
